# baseline (speedup 1.0000x reference)
import jax
import jax.numpy as jnp
from jax.experimental import pallas as pl
from jax.experimental.pallas import tpu as pltpu

_G = 10


def _body(ctx_ref, v_ref, e_ref, w_ref, b_ref, o_ref, vacc, eacc):
    i = pl.program_id(0)

    @pl.when(i == 0)
    def _init():
        vacc[...] = jnp.zeros_like(vacc)
        eacc[...] = jnp.zeros_like(eacc)

    vacc[...] += jnp.sum(v_ref[...], axis=0, keepdims=True)
    eacc[...] += jnp.sum(e_ref[...], axis=0, keepdims=True)

    @pl.when(i == _G - 1)
    def _finish():
        n_v = v_ref.shape[0] * _G
        n_e = e_ref.shape[0] * _G * 8
        v_agg = vacc[...] / n_v
        e_agg = eacc[...] / n_e
        out = jnp.dot(ctx_ref[...], w_ref[0:128],
                      preferred_element_type=jnp.float32)
        out += jnp.dot(v_agg, w_ref[128:256],
                       preferred_element_type=jnp.float32)
        out += jnp.dot(e_agg, w_ref[256:384],
                       preferred_element_type=jnp.float32)
        o_ref[...] = out + b_ref[...]


def kernel(context, vertex_data, edge_data, W, b):
    n_verts, d_feat = vertex_data.shape
    n_edges, d_edge = edge_data.shape
    d_ctx = context.shape[0]

    e_packed = edge_data.reshape(n_edges // 8, 128)
    w_stack = jnp.concatenate(
        [W[:d_ctx], W[d_ctx:d_ctx + d_feat], jnp.tile(W[d_ctx + d_feat:], (8, 1))],
        axis=0,
    )

    vc = n_verts // _G
    ec = (n_edges // 8) // _G

    out = pl.pallas_call(
        _body,
        grid=(_G,),
        in_specs=[
            pl.BlockSpec((1, d_ctx), lambda i: (0, 0)),
            pl.BlockSpec((vc, d_feat), lambda i: (i, 0)),
            pl.BlockSpec((ec, 128), lambda i: (i, 0)),
            pl.BlockSpec((384, d_ctx), lambda i: (0, 0)),
            pl.BlockSpec((1, d_ctx), lambda i: (0, 0)),
        ],
        out_specs=pl.BlockSpec((1, d_ctx), lambda i: (0, 0)),
        out_shape=jax.ShapeDtypeStruct((1, d_ctx), jnp.float32),
        scratch_shapes=[
            pltpu.VMEM((1, d_feat), jnp.float32),
            pltpu.VMEM((1, 128), jnp.float32),
        ],
    )(context.reshape(1, d_ctx), vertex_data, e_packed, w_stack,
      b.reshape(1, d_ctx))

    return out.reshape(d_ctx)
